# baseline (speedup 1.0000x reference)
import jax
import jax.numpy as jnp
from jax import lax
from jax.experimental import pallas as pl
from jax.experimental.pallas import tpu as pltpu
from jax.experimental.pallas import tpu_sc as plsc

B = 16384
N_CAT = 26
N_CONT = 16
N_TOK = N_CAT + N_CONT
CARD = 100000
DIM = 32
V = N_CAT * CARD

NC = 2
NS = 16
NW = NC * NS

TL = 512
TB = V // TL
TAIL = V - TB * TL
TB_W = (TB + NW - 1) // NW

GB = 128
N_GRP = B // GB
GRP_W = N_GRP // NW
CB = 16
NCH = GB // CB
BAND = 14
BROWS = BAND * DIM
NF0 = 14
NF1 = 12
SEG = GB * N_CAT


def _xpose_body(t32_hbm, tail_hbm, t4_hbm, inb, outb, tailb, isem, osem):
    wid = lax.axis_index("s") * NC + lax.axis_index("c")
    iota = lax.iota(jnp.int32, 16)

    def rows(src, nrow):
        def row(g, carry2):
            for m in range(8):
                dv = iota + (m % 2) * 16
                cv = jnp.full((16,), 4 * g + m // 2, jnp.int32)
                outb[g, pl.ds(m * 16, 16)] = plsc.load_gather(src, [dv, cv])
            return carry2

        lax.fori_loop(0, nrow, row, 0)

    def block(i, carry):
        bid = i * NW + wid

        @pl.when(bid < TB)
        def _():
            c0 = pl.multiple_of(bid * TL, TL)
            pltpu.async_copy(t32_hbm.at[:, pl.ds(c0, TL)], inb, isem).wait()
            rows(inb, TL // 4)
            pltpu.async_copy(
                outb, t4_hbm.at[pl.ds(pl.multiple_of(c0 // 4, TL // 4),
                                      TL // 4)], osem).wait()
        return carry

    lax.fori_loop(0, TB_W, block, 0)

    @pl.when(wid == 0)
    def _():
        pltpu.sync_copy(tail_hbm, tailb)
        rows(tailb, TAIL // 4)
        pltpu.async_copy(outb.at[pl.ds(0, TAIL // 4)],
                         t4_hbm.at[pl.ds(TB * TL // 4, TAIL // 4)],
                         osem).wait()


def _gather_body(gidx4_hbm, qoff_hbm, xt_hbm, wb_hbm, t4_hbm,
                 out_hbm,
                 idx_v, qoff_v, wide_v, stage_v, xv, wbv, gsem):
    wid = lax.axis_index("s") * NC + lax.axis_index("c")
    iota = lax.iota(jnp.int32, 16)

    pltpu.sync_copy(wb_hbm, wbv)

    def cat_band(seg0, nf, f0):
        def chunk(c, carry):
            nrow = CB * nf
            r0 = seg0 + c * nrow
            pltpu.sync_copy(gidx4_hbm.at[pl.ds(r0, nrow)],
                            idx_v.at[pl.ds(0, nrow)])
            pltpu.sync_copy(qoff_hbm.at[pl.ds(r0, nrow)],
                            qoff_v.at[pl.ds(0, nrow)])
            pltpu.async_copy(t4_hbm.at[idx_v.at[pl.ds(0, nrow)]],
                             wide_v.at[pl.ds(0, nrow)], gsem).wait()
            for fl in range(nf):
                i_vec = iota * nf + fl
                q_vec = plsc.load_gather(qoff_v, [i_vec])
                for d in range(DIM):
                    vals = plsc.load_gather(wide_v, [i_vec, q_vec + d])
                    stage_v[(fl + f0) * DIM + d, pl.ds(c * CB, CB)] = vals
            return carry

        lax.fori_loop(0, NCH, chunk, 0)

    def cont_rows(fc, row0):
        w0 = wbv[pl.ds(fc * DIM, 16)]
        w1 = wbv[pl.ds(fc * DIM + 16, 16)]
        bias0 = wbv[pl.ds((N_CONT + fc) * DIM, 16)]
        bias1 = wbv[pl.ds((N_CONT + fc) * DIM + 16, 16)]

        def lanes(lg, carry):
            xr = xv[fc, pl.ds(lg * 16, 16)]
            for d in range(DIM):
                ws = w0[d] if d < 16 else w1[d - 16]
                bs = bias0[d] if d < 16 else bias1[d - 16]
                stage_v[row0 + d, pl.ds(lg * 16, 16)] = xr * ws + bs
            return carry

        lax.fori_loop(0, NCH, lanes, 0)

    def group(g, carry):
        gg = wid * GRP_W + g
        b0 = pl.multiple_of(gg * GB, GB)
        seg0 = gg * SEG
        pltpu.sync_copy(xt_hbm.at[:, pl.ds(b0, GB)], xv)

        cat_band(seg0, NF0, 0)
        pltpu.sync_copy(stage_v, out_hbm.at[pl.ds(0, BROWS),
                                            pl.ds(b0, GB)])
        cat_band(seg0 + GB * NF0, NF1, 0)
        cont_rows(0, NF1 * DIM)
        cont_rows(1, (NF1 + 1) * DIM)
        pltpu.sync_copy(stage_v, out_hbm.at[pl.ds(BROWS, BROWS),
                                            pl.ds(b0, GB)])
        for fc in range(2, N_CONT):
            cont_rows(fc, (fc - 2) * DIM)
        pltpu.sync_copy(stage_v, out_hbm.at[pl.ds(2 * BROWS, BROWS),
                                            pl.ds(b0, GB)])
        return carry

    lax.fori_loop(0, GRP_W, group, 0)


@jax.jit
def kernel(x_cat, x_cont, cat_table, cont_W, cont_b):
    t32 = cat_table.T
    tail = cat_table[V - TAIL:].T
    xt = x_cont.T
    offsets = jnp.arange(N_CAT, dtype=jnp.int32) * CARD
    flat = x_cat.astype(jnp.int32) + offsets[None, :]
    a0 = flat[:, :NF0].reshape(N_GRP, GB * NF0)
    a1 = flat[:, NF0:].reshape(N_GRP, GB * NF1)
    ordered = jnp.concatenate([a0, a1], axis=1).reshape(-1)
    gidx4 = ordered >> 2
    qoff = (ordered & 3) * DIM
    wb = jnp.concatenate([cont_W.reshape(-1), cont_b.reshape(-1)])

    mesh = plsc.VectorSubcoreMesh(core_axis_name="c", subcore_axis_name="s",
                                  num_cores=NC, num_subcores=NS)
    params = pltpu.CompilerParams(use_tc_tiling_on_sc=True,
                                  needs_layout_passes=False)

    t4 = pl.kernel(
        _xpose_body,
        out_type=jax.ShapeDtypeStruct((V // 4, 128), jnp.float32),
        mesh=mesh,
        scratch_types=[
            pltpu.VMEM((32, TL), jnp.float32),
            pltpu.VMEM((TL // 4, 128), jnp.float32),
            pltpu.VMEM((32, TAIL), jnp.float32),
            pltpu.SemaphoreType.DMA,
            pltpu.SemaphoreType.DMA,
        ],
        compiler_params=params,
    )(t32, tail)

    out_p = pl.kernel(
        _gather_body,
        out_type=jax.ShapeDtypeStruct((N_TOK * DIM, B), jnp.float32),
        mesh=mesh,
        scratch_types=[
            pltpu.VMEM((CB * NF0,), jnp.int32),
            pltpu.VMEM((CB * NF0,), jnp.int32),
            pltpu.VMEM((CB * NF0, 128), jnp.float32),
            pltpu.VMEM((BROWS, GB), jnp.float32),
            pltpu.VMEM((N_CONT, GB), jnp.float32),
            pltpu.VMEM((2 * N_CONT * DIM,), jnp.float32),
            pltpu.SemaphoreType.DMA,
        ],
        compiler_params=params,
    )(gidx4, qoff, xt, wb, t4)
    return out_p.reshape(N_TOK, DIM, B).transpose(2, 0, 1)


# speedup vs baseline: 1.0462x; 1.0462x over previous
import jax
import jax.numpy as jnp
from jax import lax
from jax.experimental import pallas as pl
from jax.experimental.pallas import tpu as pltpu
from jax.experimental.pallas import tpu_sc as plsc

B = 16384
N_CAT = 26
N_CONT = 16
N_TOK = N_CAT + N_CONT
CARD = 100000
DIM = 32
V = N_CAT * CARD

NC = 2
NS = 16
NW = NC * NS

TL = 512
TB = V // TL
TAIL = V - TB * TL
TB_W = (TB + NW - 1) // NW

GB = 128
N_GRP = B // GB
GRP_W = N_GRP // NW
CB = 16
NCH = GB // CB
BAND = 14
BROWS = BAND * DIM
NF0 = 14
NF1 = 12
SEG = GB * N_CAT


def _xpose_body(t32_hbm, tail_hbm, t4_hbm,
                inb0, inb1, outb0, outb1, tailb,
                isem0, isem1, osem0, osem1):
    wid = lax.axis_index("s") * NC + lax.axis_index("c")
    iota = lax.iota(jnp.int32, 16)
    dv0 = iota
    dv1 = iota + 16

    def rows(src, dst, nrow):
        def row(g2, carry2):
            for gg in range(2):
                g = g2 * 2 + gg
                for half in range(4):
                    cv = jnp.full((16,), 4 * g + half, jnp.int32)
                    dst[g, pl.ds(half * 32, 16)] = plsc.load_gather(
                        src, [dv0, cv])
                    dst[g, pl.ds(half * 32 + 16, 16)] = plsc.load_gather(
                        src, [dv1, cv])
            return carry2

        lax.fori_loop(0, nrow // 2, row, 0)

    def in_slice(bid):
        c0 = pl.multiple_of(jnp.minimum(bid, TB - 1) * TL, TL)
        return t32_hbm.at[:, pl.ds(c0, TL)]

    def out_slice(bid):
        r0 = pl.multiple_of(jnp.minimum(bid, TB - 1) * (TL // 4), TL // 4)
        return t4_hbm.at[pl.ds(r0, TL // 4)]

    def pair(p, carry):
        b0 = (2 * p) * NW + wid
        b1 = (2 * p + 1) * NW + wid
        d0 = pltpu.async_copy(in_slice(b0), inb0, isem0)
        d1 = pltpu.async_copy(in_slice(b1), inb1, isem1)
        d0.wait()
        rows(inb0, outb0, TL // 4)
        o0 = pltpu.async_copy(outb0, out_slice(b0), osem0)
        d1.wait()
        rows(inb1, outb1, TL // 4)
        o1 = pltpu.async_copy(outb1, out_slice(b1), osem1)
        o0.wait()
        o1.wait()
        return carry

    lax.fori_loop(0, (TB_W + 1) // 2, pair, 0)

    @pl.when(wid == 0)
    def _():
        pltpu.sync_copy(tail_hbm, tailb)
        rows(tailb, outb0, TAIL // 4)
        pltpu.async_copy(outb0.at[pl.ds(0, TAIL // 4)],
                         t4_hbm.at[pl.ds(TB * TL // 4, TAIL // 4)],
                         osem0).wait()


def _gather_body(gidx4_hbm, qoff_hbm, xt_hbm, wb_hbm, t4_hbm,
                 out_hbm,
                 idx_v, qoff_v, wide_v, stage_v, xv, wbv, gsem):
    wid = lax.axis_index("s") * NC + lax.axis_index("c")
    iota = lax.iota(jnp.int32, 16)

    pltpu.sync_copy(wb_hbm, wbv)

    def cat_band(seg0, nf, f0):
        def chunk(c, carry):
            nrow = CB * nf
            r0 = seg0 + c * nrow
            pltpu.sync_copy(gidx4_hbm.at[pl.ds(r0, nrow)],
                            idx_v.at[pl.ds(0, nrow)])
            pltpu.sync_copy(qoff_hbm.at[pl.ds(r0, nrow)],
                            qoff_v.at[pl.ds(0, nrow)])
            pltpu.async_copy(t4_hbm.at[idx_v.at[pl.ds(0, nrow)]],
                             wide_v.at[pl.ds(0, nrow)], gsem).wait()
            for fl in range(nf):
                i_vec = iota * nf + fl
                q_vec = plsc.load_gather(qoff_v, [i_vec])
                for d in range(DIM):
                    vals = plsc.load_gather(wide_v, [i_vec, q_vec + d])
                    stage_v[(fl + f0) * DIM + d, pl.ds(c * CB, CB)] = vals
            return carry

        lax.fori_loop(0, NCH, chunk, 0)

    def cont_rows(fc, row0):
        w0 = wbv[pl.ds(fc * DIM, 16)]
        w1 = wbv[pl.ds(fc * DIM + 16, 16)]
        bias0 = wbv[pl.ds((N_CONT + fc) * DIM, 16)]
        bias1 = wbv[pl.ds((N_CONT + fc) * DIM + 16, 16)]

        def lanes(lg, carry):
            xr = xv[fc, pl.ds(lg * 16, 16)]
            for d in range(DIM):
                ws = w0[d] if d < 16 else w1[d - 16]
                bs = bias0[d] if d < 16 else bias1[d - 16]
                stage_v[row0 + d, pl.ds(lg * 16, 16)] = xr * ws + bs
            return carry

        lax.fori_loop(0, NCH, lanes, 0)

    def group(g, carry):
        gg = wid * GRP_W + g
        b0 = pl.multiple_of(gg * GB, GB)
        seg0 = gg * SEG
        pltpu.sync_copy(xt_hbm.at[:, pl.ds(b0, GB)], xv)

        cat_band(seg0, NF0, 0)
        pltpu.sync_copy(stage_v, out_hbm.at[pl.ds(0, BROWS),
                                            pl.ds(b0, GB)])
        cat_band(seg0 + GB * NF0, NF1, 0)
        cont_rows(0, NF1 * DIM)
        cont_rows(1, (NF1 + 1) * DIM)
        pltpu.sync_copy(stage_v, out_hbm.at[pl.ds(BROWS, BROWS),
                                            pl.ds(b0, GB)])
        for fc in range(2, N_CONT):
            cont_rows(fc, (fc - 2) * DIM)
        pltpu.sync_copy(stage_v, out_hbm.at[pl.ds(2 * BROWS, BROWS),
                                            pl.ds(b0, GB)])
        return carry

    lax.fori_loop(0, GRP_W, group, 0)


@jax.jit
def kernel(x_cat, x_cont, cat_table, cont_W, cont_b):
    t32 = cat_table.T
    tail = cat_table[V - TAIL:].T
    xt = x_cont.T
    offsets = jnp.arange(N_CAT, dtype=jnp.int32) * CARD
    flat = x_cat.astype(jnp.int32) + offsets[None, :]
    a0 = flat[:, :NF0].reshape(N_GRP, GB * NF0)
    a1 = flat[:, NF0:].reshape(N_GRP, GB * NF1)
    ordered = jnp.concatenate([a0, a1], axis=1).reshape(-1)
    gidx4 = ordered >> 2
    qoff = (ordered & 3) * DIM
    wb = jnp.concatenate([cont_W.reshape(-1), cont_b.reshape(-1)])

    mesh = plsc.VectorSubcoreMesh(core_axis_name="c", subcore_axis_name="s",
                                  num_cores=NC, num_subcores=NS)
    params = pltpu.CompilerParams(use_tc_tiling_on_sc=True,
                                  needs_layout_passes=False)

    t4 = pl.kernel(
        _xpose_body,
        out_type=jax.ShapeDtypeStruct((V // 4, 128), jnp.float32),
        mesh=mesh,
        scratch_types=[
            pltpu.VMEM((32, TL), jnp.float32),
            pltpu.VMEM((32, TL), jnp.float32),
            pltpu.VMEM((TL // 4, 128), jnp.float32),
            pltpu.VMEM((TL // 4, 128), jnp.float32),
            pltpu.VMEM((32, TAIL), jnp.float32),
            pltpu.SemaphoreType.DMA,
            pltpu.SemaphoreType.DMA,
            pltpu.SemaphoreType.DMA,
            pltpu.SemaphoreType.DMA,
        ],
        compiler_params=params,
    )(t32, tail)

    out_p = pl.kernel(
        _gather_body,
        out_type=jax.ShapeDtypeStruct((N_TOK * DIM, B), jnp.float32),
        mesh=mesh,
        scratch_types=[
            pltpu.VMEM((CB * NF0,), jnp.int32),
            pltpu.VMEM((CB * NF0,), jnp.int32),
            pltpu.VMEM((CB * NF0, 128), jnp.float32),
            pltpu.VMEM((BROWS, GB), jnp.float32),
            pltpu.VMEM((N_CONT, GB), jnp.float32),
            pltpu.VMEM((2 * N_CONT * DIM,), jnp.float32),
            pltpu.SemaphoreType.DMA,
        ],
        compiler_params=params,
    )(gidx4, qoff, xt, wb, t4)
    return out_p.reshape(N_TOK, DIM, B).transpose(2, 0, 1)


# speedup vs baseline: 1.2118x; 1.1583x over previous
import jax
import jax.numpy as jnp
from jax import lax
from jax.experimental import pallas as pl
from jax.experimental.pallas import tpu as pltpu
from jax.experimental.pallas import tpu_sc as plsc

B = 16384
N_CAT = 26
N_CONT = 16
N_TOK = N_CAT + N_CONT
CARD = 100000
DIM = 32
V = N_CAT * CARD

NC = 2
NS = 16
NW = NC * NS

TL = 512
TB = V // TL
TAIL = V - TB * TL
TB_W = (TB + NW - 1) // NW

GB = 128
N_GRP = B // GB
GRP_W = N_GRP // NW
CB = 16
NCH = GB // CB
BAND = 14
BROWS = BAND * DIM
NF0 = 14
NF1 = 12
SEG = GB * N_CAT


def _xpose_body(t32_hbm, tail_hbm, t4_hbm,
                inb0, inb1, outb0, outb1, tailb,
                isem0, isem1, osem0, osem1):
    wid = lax.axis_index("s") * NC + lax.axis_index("c")
    iota = lax.iota(jnp.int32, 16)
    pat = (iota // 4) * 128 + (iota % 4) * DIM

    rpat = iota // 4
    cpat = (iota % 4) * DIM

    def rows(src, dst, nrow):
        def cgrp(cg, carry2):
            rv = rpat + cg * 4
            for d in range(DIM):
                vals = src[d, pl.ds(cg * 16, 16)]
                plsc.store_scatter(dst, [rv, cpat + d], vals)
            return carry2

        lax.fori_loop(0, nrow * 4 // 16, cgrp, 0)

    def in_slice(bid):
        c0 = pl.multiple_of(jnp.minimum(bid, TB - 1) * TL, TL)
        return t32_hbm.at[:, pl.ds(c0, TL)]

    def out_slice(bid):
        r0 = pl.multiple_of(jnp.minimum(bid, TB - 1) * (TL // 4), TL // 4)
        return t4_hbm.at[pl.ds(r0, TL // 4)]

    def pair(p, carry):
        b0 = (2 * p) * NW + wid
        b1 = (2 * p + 1) * NW + wid
        d0 = pltpu.async_copy(in_slice(b0), inb0, isem0)
        d1 = pltpu.async_copy(in_slice(b1), inb1, isem1)
        d0.wait()
        rows(inb0, outb0, TL // 4)
        o0 = pltpu.async_copy(outb0, out_slice(b0), osem0)
        d1.wait()
        rows(inb1, outb1, TL // 4)
        o1 = pltpu.async_copy(outb1, out_slice(b1), osem1)
        o0.wait()
        o1.wait()
        return carry

    lax.fori_loop(0, (TB_W + 1) // 2, pair, 0)

    @pl.when(wid == 0)
    def _():
        pltpu.sync_copy(tail_hbm, tailb)
        rows(tailb, outb0, TAIL // 4)
        pltpu.async_copy(outb0.at[pl.ds(0, TAIL // 4)],
                         t4_hbm.at[pl.ds(TB * TL // 4, TAIL // 4)],
                         osem0).wait()


def _gather_body(gidx4_hbm, qoff_hbm, xt_hbm, wb_hbm, t4_hbm,
                 out_hbm,
                 idx_v, qoff_v, wide_v, stage_v, xv, wbv, gsem):
    wid = lax.axis_index("s") * NC + lax.axis_index("c")
    iota = lax.iota(jnp.int32, 16)

    pltpu.sync_copy(wb_hbm, wbv)

    def cat_band(seg0, nf, f0):
        def chunk(c, carry):
            nrow = CB * nf
            r0 = seg0 + c * nrow
            pltpu.sync_copy(gidx4_hbm.at[pl.ds(r0, nrow)],
                            idx_v.at[pl.ds(0, nrow)])
            pltpu.sync_copy(qoff_hbm.at[pl.ds(r0, nrow)],
                            qoff_v.at[pl.ds(0, nrow)])
            pltpu.async_copy(t4_hbm.at[idx_v.at[pl.ds(0, nrow)]],
                             wide_v.at[pl.ds(0, nrow)], gsem).wait()
            for fl in range(nf):
                i_vec = iota * nf + fl
                q_vec = plsc.load_gather(qoff_v, [i_vec])
                for d in range(DIM):
                    vals = plsc.load_gather(wide_v, [i_vec, q_vec + d])
                    stage_v[(fl + f0) * DIM + d, pl.ds(c * CB, CB)] = vals
            return carry

        lax.fori_loop(0, NCH, chunk, 0)

    def cont_rows(fc, row0):
        w0 = wbv[pl.ds(fc * DIM, 16)]
        w1 = wbv[pl.ds(fc * DIM + 16, 16)]
        bias0 = wbv[pl.ds((N_CONT + fc) * DIM, 16)]
        bias1 = wbv[pl.ds((N_CONT + fc) * DIM + 16, 16)]

        def lanes(lg, carry):
            xr = xv[fc, pl.ds(lg * 16, 16)]
            for d in range(DIM):
                ws = w0[d] if d < 16 else w1[d - 16]
                bs = bias0[d] if d < 16 else bias1[d - 16]
                stage_v[row0 + d, pl.ds(lg * 16, 16)] = xr * ws + bs
            return carry

        lax.fori_loop(0, NCH, lanes, 0)

    def group(g, carry):
        gg = wid * GRP_W + g
        b0 = pl.multiple_of(gg * GB, GB)
        seg0 = gg * SEG
        pltpu.sync_copy(xt_hbm.at[:, pl.ds(b0, GB)], xv)

        cat_band(seg0, NF0, 0)
        pltpu.sync_copy(stage_v, out_hbm.at[pl.ds(0, BROWS),
                                            pl.ds(b0, GB)])
        cat_band(seg0 + GB * NF0, NF1, 0)
        cont_rows(0, NF1 * DIM)
        cont_rows(1, (NF1 + 1) * DIM)
        pltpu.sync_copy(stage_v, out_hbm.at[pl.ds(BROWS, BROWS),
                                            pl.ds(b0, GB)])
        for fc in range(2, N_CONT):
            cont_rows(fc, (fc - 2) * DIM)
        pltpu.sync_copy(stage_v, out_hbm.at[pl.ds(2 * BROWS, BROWS),
                                            pl.ds(b0, GB)])
        return carry

    lax.fori_loop(0, GRP_W, group, 0)


@jax.jit
def kernel(x_cat, x_cont, cat_table, cont_W, cont_b):
    t32 = cat_table.T
    tail = cat_table[V - TAIL:].T
    xt = x_cont.T
    offsets = jnp.arange(N_CAT, dtype=jnp.int32) * CARD
    flat = x_cat.astype(jnp.int32) + offsets[None, :]
    a0 = flat[:, :NF0].reshape(N_GRP, GB * NF0)
    a1 = flat[:, NF0:].reshape(N_GRP, GB * NF1)
    ordered = jnp.concatenate([a0, a1], axis=1).reshape(-1)
    gidx4 = ordered >> 2
    qoff = (ordered & 3) * DIM
    wb = jnp.concatenate([cont_W.reshape(-1), cont_b.reshape(-1)])

    mesh = plsc.VectorSubcoreMesh(core_axis_name="c", subcore_axis_name="s",
                                  num_cores=NC, num_subcores=NS)
    params = pltpu.CompilerParams(use_tc_tiling_on_sc=True,
                                  needs_layout_passes=False)

    t4 = pl.kernel(
        _xpose_body,
        out_type=jax.ShapeDtypeStruct((V // 4, 128), jnp.float32),
        mesh=mesh,
        scratch_types=[
            pltpu.VMEM((32, TL), jnp.float32),
            pltpu.VMEM((32, TL), jnp.float32),
            pltpu.VMEM((TL // 4, 128), jnp.float32),
            pltpu.VMEM((TL // 4, 128), jnp.float32),
            pltpu.VMEM((32, TAIL), jnp.float32),
            pltpu.SemaphoreType.DMA,
            pltpu.SemaphoreType.DMA,
            pltpu.SemaphoreType.DMA,
            pltpu.SemaphoreType.DMA,
        ],
        compiler_params=params,
    )(t32, tail)

    out_p = pl.kernel(
        _gather_body,
        out_type=jax.ShapeDtypeStruct((N_TOK * DIM, B), jnp.float32),
        mesh=mesh,
        scratch_types=[
            pltpu.VMEM((CB * NF0,), jnp.int32),
            pltpu.VMEM((CB * NF0,), jnp.int32),
            pltpu.VMEM((CB * NF0, 128), jnp.float32),
            pltpu.VMEM((BROWS, GB), jnp.float32),
            pltpu.VMEM((N_CONT, GB), jnp.float32),
            pltpu.VMEM((2 * N_CONT * DIM,), jnp.float32),
            pltpu.SemaphoreType.DMA,
        ],
        compiler_params=params,
    )(gidx4, qoff, xt, wb, t4)
    return out_p.reshape(N_TOK, DIM, B).transpose(2, 0, 1)


# speedup vs baseline: 1.4097x; 1.1633x over previous
import jax
import jax.numpy as jnp
from jax import lax
from jax.experimental import pallas as pl
from jax.experimental.pallas import tpu as pltpu
from jax.experimental.pallas import tpu_sc as plsc

B = 16384
N_CAT = 26
N_CONT = 16
N_TOK = N_CAT + N_CONT
CARD = 100000
DIM = 32
V = N_CAT * CARD

NC = 2
NS = 16
NW = NC * NS

TL = 512
TB = V // TL
TAIL = V - TB * TL
TB_W = (TB + NW - 1) // NW

GB = 128
N_GRP = B // GB
GRP_W = N_GRP // NW
CB = 16
NCH = GB // CB
BAND = 14
BROWS = BAND * DIM
NF0 = 14
NF1 = 12
SEG = GB * N_CAT


def _xpose_body(t32_hbm, tail_hbm, t4_hbm,
                inb0, inb1, outb0, outb1, tailb,
                isem0, isem1, osem0, osem1):
    wid = lax.axis_index("s") * NC + lax.axis_index("c")
    iota = lax.iota(jnp.int32, 16)
    pat = (iota // 4) * 128 + (iota % 4) * DIM

    rpat = iota // 4
    cpat = (iota % 4) * DIM

    def rows(src, dst, nrow):
        @plsc.parallel_loop(0, nrow * 4 // 16, unroll=4)
        def cgrp(cg):
            rv = rpat + cg * 4
            for d in range(DIM):
                vals = src[d, pl.ds(cg * 16, 16)]
                plsc.store_scatter(dst, [rv, cpat + d], vals)

    def in_slice(bid):
        c0 = pl.multiple_of(jnp.minimum(bid, TB - 1) * TL, TL)
        return t32_hbm.at[:, pl.ds(c0, TL)]

    def out_slice(bid):
        r0 = pl.multiple_of(jnp.minimum(bid, TB - 1) * (TL // 4), TL // 4)
        return t4_hbm.at[pl.ds(r0, TL // 4)]

    def pair(p, carry):
        b0 = (2 * p) * NW + wid
        b1 = (2 * p + 1) * NW + wid
        d0 = pltpu.async_copy(in_slice(b0), inb0, isem0)
        d1 = pltpu.async_copy(in_slice(b1), inb1, isem1)
        d0.wait()
        rows(inb0, outb0, TL // 4)
        o0 = pltpu.async_copy(outb0, out_slice(b0), osem0)
        d1.wait()
        rows(inb1, outb1, TL // 4)
        o1 = pltpu.async_copy(outb1, out_slice(b1), osem1)
        o0.wait()
        o1.wait()
        return carry

    lax.fori_loop(0, (TB_W + 1) // 2, pair, 0)

    @pl.when(wid == 0)
    def _():
        pltpu.sync_copy(tail_hbm, tailb)
        rows(tailb, outb0, TAIL // 4)
        pltpu.async_copy(outb0.at[pl.ds(0, TAIL // 4)],
                         t4_hbm.at[pl.ds(TB * TL // 4, TAIL // 4)],
                         osem0).wait()


def _gather_body(gidx4_hbm, qoff_hbm, xt_hbm, wb_hbm, t4_hbm,
                 out_hbm,
                 idx_v, qoff_v, wide_v, stage_v, xv, wbv, gsem):
    wid = lax.axis_index("s") * NC + lax.axis_index("c")
    iota = lax.iota(jnp.int32, 16)

    pltpu.sync_copy(wb_hbm, wbv)

    def cat_band(seg0, nf, f0):
        def chunk(c, carry):
            nrow = CB * nf
            r0 = seg0 + c * nrow
            pltpu.sync_copy(gidx4_hbm.at[pl.ds(r0, nrow)],
                            idx_v.at[pl.ds(0, nrow)])
            pltpu.sync_copy(qoff_hbm.at[pl.ds(r0, nrow)],
                            qoff_v.at[pl.ds(0, nrow)])
            pltpu.async_copy(t4_hbm.at[idx_v.at[pl.ds(0, nrow)]],
                             wide_v.at[pl.ds(0, nrow)], gsem).wait()
            for fl in range(nf):
                i_vec = iota * nf + fl
                q_vec = plsc.load_gather(qoff_v, [i_vec])
                for d in range(DIM):
                    vals = plsc.load_gather(wide_v, [i_vec, q_vec + d])
                    stage_v[(fl + f0) * DIM + d, pl.ds(c * CB, CB)] = vals
            return carry

        lax.fori_loop(0, NCH, chunk, 0)

    def cont_rows(fc, row0):
        w0 = wbv[pl.ds(fc * DIM, 16)]
        w1 = wbv[pl.ds(fc * DIM + 16, 16)]
        bias0 = wbv[pl.ds((N_CONT + fc) * DIM, 16)]
        bias1 = wbv[pl.ds((N_CONT + fc) * DIM + 16, 16)]

        def lanes(lg, carry):
            xr = xv[fc, pl.ds(lg * 16, 16)]
            for d in range(DIM):
                ws = w0[d] if d < 16 else w1[d - 16]
                bs = bias0[d] if d < 16 else bias1[d - 16]
                stage_v[row0 + d, pl.ds(lg * 16, 16)] = xr * ws + bs
            return carry

        lax.fori_loop(0, NCH, lanes, 0)

    def group(g, carry):
        gg = wid * GRP_W + g
        b0 = pl.multiple_of(gg * GB, GB)
        seg0 = gg * SEG
        pltpu.sync_copy(xt_hbm.at[:, pl.ds(b0, GB)], xv)

        cat_band(seg0, NF0, 0)
        pltpu.sync_copy(stage_v, out_hbm.at[pl.ds(0, BROWS),
                                            pl.ds(b0, GB)])
        cat_band(seg0 + GB * NF0, NF1, 0)
        cont_rows(0, NF1 * DIM)
        cont_rows(1, (NF1 + 1) * DIM)
        pltpu.sync_copy(stage_v, out_hbm.at[pl.ds(BROWS, BROWS),
                                            pl.ds(b0, GB)])
        for fc in range(2, N_CONT):
            cont_rows(fc, (fc - 2) * DIM)
        pltpu.sync_copy(stage_v, out_hbm.at[pl.ds(2 * BROWS, BROWS),
                                            pl.ds(b0, GB)])
        return carry

    lax.fori_loop(0, GRP_W, group, 0)


@jax.jit
def kernel(x_cat, x_cont, cat_table, cont_W, cont_b):
    t32 = cat_table.T
    tail = cat_table[V - TAIL:].T
    xt = x_cont.T
    offsets = jnp.arange(N_CAT, dtype=jnp.int32) * CARD
    flat = x_cat.astype(jnp.int32) + offsets[None, :]
    a0 = flat[:, :NF0].reshape(N_GRP, GB * NF0)
    a1 = flat[:, NF0:].reshape(N_GRP, GB * NF1)
    ordered = jnp.concatenate([a0, a1], axis=1).reshape(-1)
    gidx4 = ordered >> 2
    qoff = (ordered & 3) * DIM
    wb = jnp.concatenate([cont_W.reshape(-1), cont_b.reshape(-1)])

    mesh = plsc.VectorSubcoreMesh(core_axis_name="c", subcore_axis_name="s",
                                  num_cores=NC, num_subcores=NS)
    params = pltpu.CompilerParams(use_tc_tiling_on_sc=True,
                                  needs_layout_passes=False)

    t4 = pl.kernel(
        _xpose_body,
        out_type=jax.ShapeDtypeStruct((V // 4, 128), jnp.float32),
        mesh=mesh,
        scratch_types=[
            pltpu.VMEM((32, TL), jnp.float32),
            pltpu.VMEM((32, TL), jnp.float32),
            pltpu.VMEM((TL // 4, 128), jnp.float32),
            pltpu.VMEM((TL // 4, 128), jnp.float32),
            pltpu.VMEM((32, TAIL), jnp.float32),
            pltpu.SemaphoreType.DMA,
            pltpu.SemaphoreType.DMA,
            pltpu.SemaphoreType.DMA,
            pltpu.SemaphoreType.DMA,
        ],
        compiler_params=params,
    )(t32, tail)

    out_p = pl.kernel(
        _gather_body,
        out_type=jax.ShapeDtypeStruct((N_TOK * DIM, B), jnp.float32),
        mesh=mesh,
        scratch_types=[
            pltpu.VMEM((CB * NF0,), jnp.int32),
            pltpu.VMEM((CB * NF0,), jnp.int32),
            pltpu.VMEM((CB * NF0, 128), jnp.float32),
            pltpu.VMEM((BROWS, GB), jnp.float32),
            pltpu.VMEM((N_CONT, GB), jnp.float32),
            pltpu.VMEM((2 * N_CONT * DIM,), jnp.float32),
            pltpu.SemaphoreType.DMA,
        ],
        compiler_params=params,
    )(gidx4, qoff, xt, wb, t4)
    return out_p.reshape(N_TOK, DIM, B).transpose(2, 0, 1)


# speedup vs baseline: 1.6821x; 1.1932x over previous
import jax
import jax.numpy as jnp
from jax import lax
from jax.experimental import pallas as pl
from jax.experimental.pallas import tpu as pltpu
from jax.experimental.pallas import tpu_sc as plsc

B = 16384
N_CAT = 26
N_CONT = 16
N_TOK = N_CAT + N_CONT
CARD = 100000
DIM = 32
V = N_CAT * CARD

NC = 2
NS = 16
NW = NC * NS

TL = 512
TB = V // TL
TAIL = V - TB * TL
TB_W = (TB + NW - 1) // NW

GB = 128
N_GRP = B // GB
GRP_W = N_GRP // NW
CB = 16
NCH = GB // CB
BAND = 14
BROWS = BAND * DIM
NF0 = 14
NF1 = 12
SEG = GB * N_CAT


def _xpose_body(t32_hbm, tail_hbm, t4_hbm,
                inb0, inb1, outb0, outb1, tailb,
                isem0, isem1, osem0, osem1):
    wid = lax.axis_index("s") * NC + lax.axis_index("c")
    iota = lax.iota(jnp.int32, 16)
    pat = (iota // 4) * 128 + (iota % 4) * DIM

    rpat = iota // 4
    cpat = (iota % 4) * DIM

    def rows(src, dst, nrow):
        @plsc.parallel_loop(0, nrow * 4 // 16, unroll=4)
        def cgrp(cg):
            rv = rpat + cg * 4
            for d in range(DIM):
                vals = src[d, pl.ds(cg * 16, 16)]
                plsc.store_scatter(dst, [rv, cpat + d], vals)

    def in_slice(bid):
        c0 = pl.multiple_of(jnp.minimum(bid, TB - 1) * TL, TL)
        return t32_hbm.at[:, pl.ds(c0, TL)]

    def out_slice(bid):
        r0 = pl.multiple_of(jnp.minimum(bid, TB - 1) * (TL // 4), TL // 4)
        return t4_hbm.at[pl.ds(r0, TL // 4)]

    def pair(p, carry):
        b0 = (2 * p) * NW + wid
        b1 = (2 * p + 1) * NW + wid
        d0 = pltpu.async_copy(in_slice(b0), inb0, isem0)
        d1 = pltpu.async_copy(in_slice(b1), inb1, isem1)
        d0.wait()
        rows(inb0, outb0, TL // 4)
        o0 = pltpu.async_copy(outb0, out_slice(b0), osem0)
        d1.wait()
        rows(inb1, outb1, TL // 4)
        o1 = pltpu.async_copy(outb1, out_slice(b1), osem1)
        o0.wait()
        o1.wait()
        return carry

    lax.fori_loop(0, (TB_W + 1) // 2, pair, 0)

    @pl.when(wid == 0)
    def _():
        pltpu.sync_copy(tail_hbm, tailb)
        rows(tailb, outb0, TAIL // 4)
        pltpu.async_copy(outb0.at[pl.ds(0, TAIL // 4)],
                         t4_hbm.at[pl.ds(TB * TL // 4, TAIL // 4)],
                         osem0).wait()


def _gather_body(gidx4_hbm, qoff_hbm, xt_hbm, wb_hbm, t4_hbm,
                 out_hbm,
                 idx_v, qoff_v, wide_v, stage_v, xv, wbv, gsem):
    wid = lax.axis_index("s") * NC + lax.axis_index("c")
    iota = lax.iota(jnp.int32, 16)

    pltpu.sync_copy(wb_hbm, wbv)

    def cat_band(seg0, nf, f0):
        def chunk(c, carry):
            nrow = CB * nf
            r0 = seg0 + c * nrow
            pltpu.sync_copy(gidx4_hbm.at[pl.ds(r0, nrow)],
                            idx_v.at[pl.ds(0, nrow)])
            pltpu.sync_copy(qoff_hbm.at[pl.ds(r0, nrow)],
                            qoff_v.at[pl.ds(0, nrow)])
            pltpu.async_copy(t4_hbm.at[idx_v.at[pl.ds(0, nrow)]],
                             wide_v.at[pl.ds(0, nrow)], gsem).wait()
            for fl in range(nf):
                i_vec = iota * nf + fl
                q_vec = plsc.load_gather(qoff_v, [i_vec])
                for d in range(DIM):
                    vals = plsc.load_gather(wide_v, [i_vec, q_vec + d])
                    stage_v[(fl + f0) * DIM + d, pl.ds(c * CB, CB)] = vals
            return carry

        lax.fori_loop(0, NCH, chunk, 0)

    def cont_rows(fc, row0):
        w0 = wbv[pl.ds(fc * DIM, 16)]
        w1 = wbv[pl.ds(fc * DIM + 16, 16)]
        bias0 = wbv[pl.ds((N_CONT + fc) * DIM, 16)]
        bias1 = wbv[pl.ds((N_CONT + fc) * DIM + 16, 16)]

        def lanes(lg, carry):
            xr = xv[fc, pl.ds(lg * 16, 16)]
            for d in range(DIM):
                ws = w0[d] if d < 16 else w1[d - 16]
                bs = bias0[d] if d < 16 else bias1[d - 16]
                stage_v[row0 + d, pl.ds(lg * 16, 16)] = xr * ws + bs
            return carry

        lax.fori_loop(0, NCH, lanes, 0)

    def group(g, carry):
        gg = wid * GRP_W + g
        b0 = pl.multiple_of(gg * GB, GB)
        seg0 = gg * SEG
        pltpu.sync_copy(xt_hbm.at[:, pl.ds(b0, GB)], xv)

        cat_band(seg0, NF0, 0)
        pltpu.sync_copy(stage_v, out_hbm.at[pl.ds(0, BROWS),
                                            pl.ds(b0, GB)])
        cat_band(seg0 + GB * NF0, NF1, 0)
        cont_rows(0, NF1 * DIM)
        cont_rows(1, (NF1 + 1) * DIM)
        pltpu.sync_copy(stage_v, out_hbm.at[pl.ds(BROWS, BROWS),
                                            pl.ds(b0, GB)])
        for fc in range(2, N_CONT):
            cont_rows(fc, (fc - 2) * DIM)
        pltpu.sync_copy(stage_v, out_hbm.at[pl.ds(2 * BROWS, BROWS),
                                            pl.ds(b0, GB)])
        return carry

    lax.fori_loop(0, GRP_W, group, 0)


@jax.jit
def kernel(x_cat, x_cont, cat_table, cont_W, cont_b):
    t32 = cat_table.T
    tail = cat_table[V - TAIL:].T
    xt = x_cont.T
    offsets = jnp.arange(N_CAT, dtype=jnp.int32) * CARD
    flat = x_cat.astype(jnp.int32) + offsets[None, :]
    a0 = flat[:, :NF0].reshape(N_GRP, GB * NF0)
    a1 = flat[:, NF0:].reshape(N_GRP, GB * NF1)
    ordered = jnp.concatenate([a0, a1], axis=1).reshape(-1)
    gidx4 = ordered >> 2
    qoff = (ordered & 3) * DIM
    wb = jnp.concatenate([cont_W.reshape(-1), cont_b.reshape(-1)])

    mesh = plsc.VectorSubcoreMesh(core_axis_name="c", subcore_axis_name="s",
                                  num_cores=NC, num_subcores=NS)
    params = pltpu.CompilerParams(use_tc_tiling_on_sc=True,
                                  needs_layout_passes=False)

    t4 = t32.reshape(32, V // 4, 4).transpose(1, 2, 0).reshape(V // 4, 128)

    out_p = pl.kernel(
        _gather_body,
        out_type=jax.ShapeDtypeStruct((N_TOK * DIM, B), jnp.float32),
        mesh=mesh,
        scratch_types=[
            pltpu.VMEM((CB * NF0,), jnp.int32),
            pltpu.VMEM((CB * NF0,), jnp.int32),
            pltpu.VMEM((CB * NF0, 128), jnp.float32),
            pltpu.VMEM((BROWS, GB), jnp.float32),
            pltpu.VMEM((N_CONT, GB), jnp.float32),
            pltpu.VMEM((2 * N_CONT * DIM,), jnp.float32),
            pltpu.SemaphoreType.DMA,
        ],
        compiler_params=params,
    )(gidx4, qoff, xt, wb, t4)
    return out_p.reshape(N_TOK, DIM, B).transpose(2, 0, 1)
